# baseline (speedup 1.0000x reference)
import functools

import jax
import jax.numpy as jnp
from jax import lax
from jax.experimental import pallas as pl
from jax.experimental.pallas import tpu as pltpu
from jax.experimental.pallas import tpu_sc as plsc

DIM = 2048
NUM_EXPERTS = 64
TOP_K = 8
N_GROUPS = 8
TOPK_GROUP = 4
GROUP_SIZE = NUM_EXPERTS // N_GROUPS
ROUTED_SCALING_FACTOR = 2.5
N_TOK = 16384

_NEG = -1e30

NW = 32
TPW = N_TOK // NW
L = 16


def _score_block(x_ref, w_ref, b_ref, sfc_ref):
    logits = jnp.dot(x_ref[:], w_ref[:], preferred_element_type=jnp.float32)
    lp = logits.T
    sfc_ref[:] = jax.nn.sigmoid(lp) + b_ref[:]


@functools.partial(jax.jit, static_argnames=("block_t",))
def _scores(x, w_t, bias, block_t=2048):
    n = x.shape[0]
    return pl.pallas_call(
        _score_block,
        grid=(n // block_t,),
        in_specs=[
            pl.BlockSpec((block_t, DIM), lambda i: (i, 0)),
            pl.BlockSpec((DIM, NUM_EXPERTS), lambda i: (0, 0)),
            pl.BlockSpec((NUM_EXPERTS, 1), lambda i: (0, 0)),
        ],
        out_specs=pl.BlockSpec((NUM_EXPERTS, block_t), lambda i: (0, i)),
        out_shape=jax.ShapeDtypeStruct((NUM_EXPERTS, n), jnp.float32),
    )(x, w_t, bias)


def _sc_route_body(sfc_hbm, brep_hbm, idx_hbm, wgt_hbm,
                   sfc_v, bias_v, idxb_v, wgtb_v):
    wid = lax.axis_index("s") * 2 + lax.axis_index("c")
    base = wid * TPW
    pltpu.sync_copy(sfc_hbm.at[:, pl.ds(base, TPW)], sfc_v)
    pltpu.sync_copy(brep_hbm, bias_v)

    negv = jnp.full((L,), _NEG, jnp.float32)

    def chunk(ci, carry):
        col0 = ci * L
        rows = [sfc_v[e, pl.ds(col0, L)] for e in range(NUM_EXPERTS)]

        gsc = []
        for g in range(N_GROUPS):
            m1 = rows[g * GROUP_SIZE]
            m2 = negv
            for r in range(1, GROUP_SIZE):
                v = rows[g * GROUP_SIZE + r]
                m2 = jnp.maximum(m2, jnp.minimum(m1, v))
                m1 = jnp.maximum(m1, v)
            gsc.append(m1 + m2)

        selg = [jnp.zeros((L,), jnp.bool_) for _ in range(N_GROUPS)]
        for _ in range(TOPK_GROUP):
            bv = negv
            bi = jnp.full((L,), N_GROUPS, jnp.int32)
            for g in range(N_GROUPS):
                better = gsc[g] > bv
                bv = jnp.where(better, gsc[g], bv)
                bi = jnp.where(better, g, bi)
            for g in range(N_GROUPS):
                hit = bi == g
                selg[g] = selg[g] | hit
                gsc[g] = jnp.where(hit, _NEG, gsc[g])

        tmp = [jnp.where(selg[e // GROUP_SIZE], rows[e], 0.0)
               for e in range(NUM_EXPERTS)]

        denom = jnp.zeros((L,), jnp.float32)
        bis = []
        ws = []
        for _ in range(TOP_K):
            bv = negv
            bi = jnp.full((L,), NUM_EXPERTS, jnp.int32)
            bsel = jnp.zeros((L,), jnp.float32)
            for e in range(NUM_EXPERTS):
                v = tmp[e]
                better = v > bv
                bv = jnp.where(better, v, bv)
                bi = jnp.where(better, e, bi)
                bsel = jnp.where(better, bias_v[pl.ds(e * L, L)], bsel)
            for e in range(NUM_EXPERTS):
                tmp[e] = jnp.where(bi == e, _NEG, tmp[e])
            w = bv - bsel
            denom = denom + w
            bis.append(bi)
            ws.append(w)

        scale = ROUTED_SCALING_FACTOR / (denom + 1e-20)
        for k in range(TOP_K):
            idxb_v[k, pl.ds(col0, L)] = bis[k]
            wgtb_v[k, pl.ds(col0, L)] = ws[k] * scale
        return carry

    lax.fori_loop(0, TPW // L, chunk, 0)

    pltpu.sync_copy(idxb_v, idx_hbm.at[:, pl.ds(base, TPW)])
    pltpu.sync_copy(wgtb_v, wgt_hbm.at[:, pl.ds(base, TPW)])


_sc_route = functools.partial(
    pl.kernel,
    mesh=plsc.VectorSubcoreMesh(core_axis_name="c", subcore_axis_name="s"),
    out_type=[
        jax.ShapeDtypeStruct((TOP_K, N_TOK), jnp.int32),
        jax.ShapeDtypeStruct((TOP_K, N_TOK), jnp.float32),
    ],
    scratch_types=[
        pltpu.VMEM((NUM_EXPERTS, TPW), jnp.float32),
        pltpu.VMEM((NUM_EXPERTS * L,), jnp.float32),
        pltpu.VMEM((TOP_K, TPW), jnp.int32),
        pltpu.VMEM((TOP_K, TPW), jnp.float32),
    ],
)(_sc_route_body)


def kernel(x, W_gate, e_score_correction_bias):
    w_t = W_gate.T
    bias = e_score_correction_bias.reshape(NUM_EXPERTS, 1)
    sfc = _scores(x, w_t, bias)
    brep = jnp.broadcast_to(bias, (NUM_EXPERTS, L)).reshape(-1)
    idx_t, wgt_t = _sc_route(sfc, brep)
    return idx_t.T, wgt_t.T


# speedup vs baseline: 2.4443x; 2.4443x over previous
import functools

import jax
import jax.numpy as jnp
import numpy as np
from jax.experimental import pallas as pl

DIM = 2048
NUM_EXPERTS = 64
TOP_K = 8
N_GROUPS = 8
TOPK_GROUP = 4
GROUP_SIZE = NUM_EXPERTS // N_GROUPS
ROUTED_SCALING_FACTOR = 2.5

_NEG = -1e30

_PERM = np.arange(NUM_EXPERTS).reshape(GROUP_SIZE, N_GROUPS).T.reshape(-1)


def _router_block(x_ref, w_ref, b_ref, idx_ref, wgt_ref):
    logits = jnp.dot(x_ref[:], w_ref[:], preferred_element_type=jnp.float32)
    lp = logits.T
    scores = jax.nn.sigmoid(lp)
    sfc = scores + b_ref[:]

    t = sfc.shape[1]

    m1 = sfc[0:N_GROUPS]
    m2 = jnp.full((N_GROUPS, t), _NEG, jnp.float32)
    for r in range(1, GROUP_SIZE):
        v = sfc[r * N_GROUPS:(r + 1) * N_GROUPS]
        m2 = jnp.maximum(m2, jnp.minimum(m1, v))
        m1 = jnp.maximum(m1, v)
    gsc = m1 + m2

    giota = jax.lax.broadcasted_iota(jnp.int32, (N_GROUPS, t), 0)
    sel = jnp.zeros((N_GROUPS, t), jnp.bool_)
    for _ in range(TOPK_GROUP):
        m = jnp.max(gsc, axis=0, keepdims=True)
        first = jnp.min(jnp.where(gsc == m, giota, N_GROUPS), axis=0,
                        keepdims=True)
        hit = giota == first
        sel = sel | hit
        gsc = jnp.where(hit, _NEG, gsc)

    tmp = jnp.concatenate(
        [jnp.where(sel, sfc[r * N_GROUPS:(r + 1) * N_GROUPS], 0.0)
         for r in range(GROUP_SIZE)], axis=0)

    srow = jax.lax.broadcasted_iota(jnp.int32, (NUM_EXPERTS, t), 0)
    eorig = ((srow << 3) & 56) | (srow >> 3)

    idx_rows = []
    wgt_rows = []
    for _ in range(TOP_K):
        m = jnp.max(tmp, axis=0, keepdims=True)
        first = jnp.min(jnp.where(tmp == m, eorig, NUM_EXPERTS), axis=0,
                        keepdims=True)
        onehot = eorig == first
        w = jnp.sum(jnp.where(onehot, scores, 0.0), axis=0, keepdims=True)
        idx_rows.append(first)
        wgt_rows.append(w)
        tmp = jnp.where(onehot, _NEG, tmp)

    idx = jnp.concatenate(idx_rows, axis=0)
    wgt = jnp.concatenate(wgt_rows, axis=0)
    denom = jnp.sum(wgt, axis=0, keepdims=True) + 1e-20
    wgt = wgt * (ROUTED_SCALING_FACTOR / denom)

    idx_ref[:] = idx.T
    wgt_ref[:] = wgt.T


@functools.partial(jax.jit, static_argnames=("block_t",))
def _run(x, w_t, bias, block_t=512):
    n = x.shape[0]
    grid = (n // block_t,)
    return pl.pallas_call(
        _router_block,
        grid=grid,
        in_specs=[
            pl.BlockSpec((block_t, DIM), lambda i: (i, 0)),
            pl.BlockSpec((DIM, NUM_EXPERTS), lambda i: (0, 0)),
            pl.BlockSpec((NUM_EXPERTS, 1), lambda i: (0, 0)),
        ],
        out_specs=[
            pl.BlockSpec((block_t, TOP_K), lambda i: (i, 0)),
            pl.BlockSpec((block_t, TOP_K), lambda i: (i, 0)),
        ],
        out_shape=[
            jax.ShapeDtypeStruct((n, TOP_K), jnp.int32),
            jax.ShapeDtypeStruct((n, TOP_K), jnp.float32),
        ],
    )(x, w_t, bias)


def kernel(x, W_gate, e_score_correction_bias):
    w_t = W_gate[_PERM].T
    bias = e_score_correction_bias[_PERM].reshape(NUM_EXPERTS, 1)
    idx, wgt = _run(x, w_t, bias, block_t=2048)
    return idx, wgt


# speedup vs baseline: 2.4477x; 1.0014x over previous
import functools

import jax
import jax.numpy as jnp
import numpy as np
from jax.experimental import pallas as pl

DIM = 2048
NUM_EXPERTS = 64
TOP_K = 8
N_GROUPS = 8
TOPK_GROUP = 4
GROUP_SIZE = NUM_EXPERTS // N_GROUPS
ROUTED_SCALING_FACTOR = 2.5

_NEG = -1e30

_PERM = np.arange(NUM_EXPERTS).reshape(GROUP_SIZE, N_GROUPS).T.reshape(-1)


def _router_block(xa_ref, xb_ref, w_ref, b_ref, idx_ref, wgt_ref):
    half = DIM // 2
    logits = (jnp.dot(xa_ref[:], w_ref[0:half], preferred_element_type=jnp.float32)
              + jnp.dot(xb_ref[:], w_ref[half:DIM], preferred_element_type=jnp.float32))
    lp = logits.T
    scores = jax.nn.sigmoid(lp)
    sfc = scores + b_ref[:]

    t = sfc.shape[1]

    m1 = sfc[0:N_GROUPS]
    m2 = jnp.full((N_GROUPS, t), _NEG, jnp.float32)
    for r in range(1, GROUP_SIZE):
        v = sfc[r * N_GROUPS:(r + 1) * N_GROUPS]
        m2 = jnp.maximum(m2, jnp.minimum(m1, v))
        m1 = jnp.maximum(m1, v)
    gsc = m1 + m2

    giota = jax.lax.broadcasted_iota(jnp.int32, (N_GROUPS, t), 0)
    sel = jnp.zeros((N_GROUPS, t), jnp.bool_)
    for _ in range(TOPK_GROUP):
        m = jnp.max(gsc, axis=0, keepdims=True)
        first = jnp.min(jnp.where(gsc == m, giota, N_GROUPS), axis=0,
                        keepdims=True)
        hit = giota == first
        sel = sel | hit
        gsc = jnp.where(hit, _NEG, gsc)

    tmp = jnp.concatenate(
        [jnp.where(sel, sfc[r * N_GROUPS:(r + 1) * N_GROUPS], 0.0)
         for r in range(GROUP_SIZE)], axis=0)

    srow = jax.lax.broadcasted_iota(jnp.int32, (NUM_EXPERTS, t), 0)
    eorig = ((srow << 3) & 56) | (srow >> 3)

    idx_rows = []
    wgt_rows = []
    for _ in range(TOP_K):
        m = jnp.max(tmp, axis=0, keepdims=True)
        first = jnp.min(jnp.where(tmp == m, eorig, NUM_EXPERTS), axis=0,
                        keepdims=True)
        onehot = eorig == first
        w = jnp.sum(jnp.where(onehot, scores, 0.0), axis=0, keepdims=True)
        idx_rows.append(first)
        wgt_rows.append(w)
        tmp = jnp.where(onehot, _NEG, tmp)

    idx = jnp.concatenate(idx_rows, axis=0)
    wgt = jnp.concatenate(wgt_rows, axis=0)
    denom = jnp.sum(wgt, axis=0, keepdims=True) + 1e-20
    wgt = wgt * (ROUTED_SCALING_FACTOR / denom)

    idx_ref[:] = idx.T
    wgt_ref[:] = wgt.T


@functools.partial(jax.jit, static_argnames=("block_t",))
def _run(x, w_t, bias, block_t=512):
    n = x.shape[0]
    grid = (n // block_t,)
    return pl.pallas_call(
        _router_block,
        grid=grid,
        in_specs=[
            pl.BlockSpec((block_t, DIM // 2), lambda i: (i, 0)),
            pl.BlockSpec((block_t, DIM // 2), lambda i: (i, 1)),
            pl.BlockSpec((DIM, NUM_EXPERTS), lambda i: (0, 0)),
            pl.BlockSpec((NUM_EXPERTS, 1), lambda i: (0, 0)),
        ],
        out_specs=[
            pl.BlockSpec((block_t, TOP_K), lambda i: (i, 0)),
            pl.BlockSpec((block_t, TOP_K), lambda i: (i, 0)),
        ],
        out_shape=[
            jax.ShapeDtypeStruct((n, TOP_K), jnp.int32),
            jax.ShapeDtypeStruct((n, TOP_K), jnp.float32),
        ],
    )(x, x, w_t, bias)


def kernel(x, W_gate, e_score_correction_bias):
    w_t = W_gate[_PERM].T
    bias = e_score_correction_bias[_PERM].reshape(NUM_EXPERTS, 1)
    idx, wgt = _run(x, w_t, bias, block_t=2048)
    return idx, wgt
